# baseline (speedup 1.0000x reference)
import jax
import jax.numpy as jnp
from jax.experimental import pallas as pl

N = 10000
E = 320000
D = 128
K = 5000


def _scale_kernel(xg_ref, vals_ref, o_ref):
    v = vals_ref[...]
    sig = 1.0 / (1.0 + jnp.exp(-v))
    o_ref[...] = xg_ref[...] * sig


def kernel(x, edge_index, W, b):
    src = edge_index[0]
    dst = edge_index[1]
    loop = jnp.arange(N, dtype=edge_index.dtype)
    src = jnp.concatenate([src, loop], axis=0)
    dst = jnp.concatenate([dst, loop], axis=0)
    deg = jnp.zeros((N,), jnp.float32).at[dst].add(1.0)
    dinv = jnp.where(deg > 0, deg ** -0.5, 0.0)
    norm = dinv[src] * dinv[dst]
    h = x @ W
    msg = h[src] * norm[:, None]
    out = jnp.zeros((N, D), jnp.float32).at[dst].add(msg)
    score = jnp.sum(out + b, axis=-1)
    vals, idx = jax.lax.top_k(score, K)
    xg = x[idx]
    x_pool = pl.pallas_call(
        _scale_kernel,
        out_shape=jax.ShapeDtypeStruct((K, D), jnp.float32),
    )(xg, vals[:, None])
    return x_pool


# speedup vs baseline: 4.0238x; 4.0238x over previous
import functools

import jax
import jax.numpy as jnp
from jax import lax
from jax.experimental import pallas as pl
from jax.experimental.pallas import tpu as pltpu
from jax.experimental.pallas import tpu_sc as plsc


def _mm_body(x_ref, w_ref, o_ref):
    o_ref[...] = jnp.dot(x_ref[...], w_ref[...],
                         preferred_element_type=jnp.float32)


def _matmul(x, W):
    N, D = x.shape
    return pl.pallas_call(
        _mm_body, out_shape=jax.ShapeDtypeStruct((N, D), jnp.float32))(x, W)


def _deg_partials(dst, n_nodes):
    E = dst.shape[0]
    NC, NS = 2, 16
    EPC = E // (NC * NS)
    mesh = plsc.VectorSubcoreMesh(core_axis_name="c", subcore_axis_name="s")

    @functools.partial(
        pl.kernel,
        out_type=jax.ShapeDtypeStruct((NC, n_nodes), jnp.float32),
        mesh=mesh,
        scratch_types=[
            pltpu.VMEM((EPC,), jnp.int32),
            pltpu.VMEM((EPC,), jnp.float32),
            pltpu.VMEM((n_nodes,), jnp.float32),
            pltpu.VMEM_SHARED((n_nodes,), jnp.float32),
            pltpu.SemaphoreType.DMA,
        ],
    )
    def deg_kernel(dst_hbm, out_hbm, idx_v, ones_v, zero_v, deg_sh, sem):
        c = lax.axis_index("c")
        s = lax.axis_index("s")
        wid = s * NC + c

        @pl.loop(0, EPC, step=16)
        def _(i):
            ones_v[pl.ds(i, 16)] = jnp.full((16,), 1.0, jnp.float32)

        @pl.when(s == 0)
        def _():
            @pl.loop(0, n_nodes, step=16)
            def _(i):
                zero_v[pl.ds(i, 16)] = jnp.zeros((16,), jnp.float32)
            pltpu.sync_copy(zero_v, deg_sh)

        plsc.subcore_barrier()
        pltpu.async_copy(dst_hbm.at[pl.ds(wid * EPC, EPC)], idx_v, sem).wait()
        pltpu.sync_copy(ones_v, deg_sh.at[idx_v], add=True)
        plsc.subcore_barrier()

        @pl.when(s == 0)
        def _():
            pltpu.sync_copy(deg_sh, out_hbm.at[c])

    return deg_kernel(dst)


def _dinv_body(p_ref, o_ref):
    deg = p_ref[0, :] + p_ref[1, :] + 1.0
    o_ref[...] = jnp.where(deg > 0, lax.rsqrt(deg), 0.0)


def _dinv(partials):
    n = partials.shape[1]
    return pl.pallas_call(
        _dinv_body, out_shape=jax.ShapeDtypeStruct((n,), jnp.float32))(partials)


def _accum_body(src_ref, dst_ref, h_ref, dinv_s_ref, dinv_v_ref, o_ref):
    pid = pl.program_id(0)
    G = pl.num_programs(0)
    C = src_ref.shape[2]

    @pl.when(pid == 0)
    def _():
        o_ref[...] = jnp.zeros_like(o_ref)

    U = 8

    def body(j, carry):
        for u in range(U):
            e = j * U + u
            s = src_ref[0, 0, e]
            d = dst_ref[0, 0, e]
            nrm = dinv_s_ref[0, s] * dinv_s_ref[0, d]
            o_ref[pl.ds(d, 1), :] += h_ref[pl.ds(s, 1), :] * nrm
        return carry

    lax.fori_loop(0, C // U, body, 0)

    @pl.when(pid == G - 1)
    def _():
        dv = dinv_v_ref[...]
        o_ref[...] += h_ref[...] * (dv * dv)


def _accumulate(src, dst, h, dinv):
    N, D = h.shape
    E = src.shape[0]
    G = 40
    C = E // G
    src3 = src.reshape(G, 1, C)
    dst3 = dst.reshape(G, 1, C)
    return pl.pallas_call(
        _accum_body,
        grid=(G,),
        in_specs=[
            pl.BlockSpec((1, 1, C), lambda i: (i, 0, 0),
                         memory_space=pltpu.SMEM),
            pl.BlockSpec((1, 1, C), lambda i: (i, 0, 0),
                         memory_space=pltpu.SMEM),
            pl.BlockSpec((N, D), lambda i: (0, 0)),
            pl.BlockSpec((1, N), lambda i: (0, 0),
                         memory_space=pltpu.SMEM),
            pl.BlockSpec((N, 1), lambda i: (0, 0)),
        ],
        out_specs=pl.BlockSpec((N, D), lambda i: (0, 0)),
        out_shape=jax.ShapeDtypeStruct((N, D), jnp.float32),
    )(src3, dst3, h, dinv.reshape(1, N), dinv.reshape(N, 1))


def _score_body(o_ref, b_ref, s_ref):
    ob = o_ref[...] + b_ref[...]
    acc = ob
    for k in range(1, 16):
        acc = acc + jnp.roll(ob, -8 * k, axis=1)
    acc = acc + jnp.roll(acc, -4, axis=1)
    acc = acc + jnp.roll(acc, -2, axis=1)
    acc = acc + jnp.roll(acc, -1, axis=1)
    s_ref[...] = acc[:, 0:1]


def _score(out, b):
    N, D = out.shape
    R = 1000
    return pl.pallas_call(
        _score_body,
        grid=(N // R,),
        in_specs=[
            pl.BlockSpec((R, D), lambda i: (i, 0)),
            pl.BlockSpec((1, D), lambda i: (0, 0)),
        ],
        out_specs=pl.BlockSpec((R, 1), lambda i: (i, 0)),
        out_shape=jax.ShapeDtypeStruct((N, 1), jnp.float32),
    )(out, b.reshape(1, D))[:, 0]


def _sort_body(k_ref, i_ref, vo_ref, io_ref):
    keys = k_ref[...]
    idx = i_ref[...]
    R, Lw = keys.shape
    n = R * Lw
    ro = lax.broadcasted_iota(jnp.int32, (R, Lw), 0)
    co = lax.broadcasted_iota(jnp.int32, (R, Lw), 1)
    pos = ro * Lw + co

    def xor_perm(a, j):
        if j < Lw:
            bit = (co & j) == 0
            return jnp.where(bit, jnp.roll(a, -j, axis=1),
                             jnp.roll(a, j, axis=1))
        m = j // Lw
        bit = (ro & m) == 0
        return jnp.where(bit, jnp.roll(a, -m, axis=0),
                         jnp.roll(a, m, axis=0))

    k = 2
    while k <= n:
        j = k // 2
        while j >= 1:
            kp = xor_perm(keys, j)
            ip = xor_perm(idx, j)
            bit = (pos & j) == 0
            desc_blk = (pos & k) == 0
            self_first = (keys > kp) | ((keys == kp) & (idx < ip))
            keep = bit == (self_first == desc_blk)
            keys = jnp.where(keep, keys, kp)
            idx = jnp.where(keep, idx, ip)
            j //= 2
        k *= 2

    vals = jnp.where(keys < 0, keys ^ jnp.int32(0x7FFFFFFF), keys)
    vo_ref[...] = lax.bitcast_convert_type(vals, jnp.float32)
    io_ref[...] = idx


def _topk_sort(score):
    N = score.shape[0]
    SN = 16384
    ki = lax.bitcast_convert_type(score, jnp.int32)
    ki = jnp.where(ki < 0, ki ^ jnp.int32(0x7FFFFFFF), ki)
    ki = jnp.concatenate(
        [ki, jnp.full((SN - N,), jnp.int32(-2147483648))]).reshape(128, 128)
    ii = jnp.concatenate(
        [jnp.arange(N, dtype=jnp.int32),
         jnp.arange(N, SN, dtype=jnp.int32)]).reshape(128, 128)
    vals, idx = pl.pallas_call(
        _sort_body,
        out_shape=(jax.ShapeDtypeStruct((128, 128), jnp.float32),
                   jax.ShapeDtypeStruct((128, 128), jnp.int32)),
    )(ki, ii)
    return vals.reshape(SN), idx.reshape(SN)


def _pool_body(idx_ref, x_ref, v_ref, o_ref):
    K = o_ref.shape[0]
    U = 4

    def body(j, carry):
        for u in range(U):
            r = j * U + u
            o_ref[pl.ds(r, 1), :] = x_ref[pl.ds(idx_ref[0, r], 1), :]
        return carry

    lax.fori_loop(0, K // U, body, 0)
    v = v_ref[...]
    sig = 1.0 / (1.0 + jnp.exp(-v))
    o_ref[...] = o_ref[...] * sig


def _pool(x, idx, vals):
    N, D = x.shape
    K = idx.shape[0]
    return pl.pallas_call(
        _pool_body,
        in_specs=[
            pl.BlockSpec((1, K), lambda: (0, 0), memory_space=pltpu.SMEM),
            pl.BlockSpec((N, D), lambda: (0, 0)),
            pl.BlockSpec((K, 1), lambda: (0, 0)),
        ],
        out_specs=pl.BlockSpec((K, D), lambda: (0, 0)),
        out_shape=jax.ShapeDtypeStruct((K, D), jnp.float32),
    )(idx.reshape(1, K), x, vals.reshape(K, 1))


def kernel(x, edge_index, W, b):
    N, D = x.shape
    K = N // 2
    src = edge_index[0]
    dst = edge_index[1]

    h = _matmul(x, W)
    partials = _deg_partials(dst, N)
    dinv = _dinv(partials)
    out = _accumulate(src, dst, h, dinv)
    score = _score(out, b)
    vals, idx = _topk_sort(score)
    return _pool(x, idx[:K], vals[:K])
